# baseline (speedup 1.0000x reference)
import jax
import jax.numpy as jnp
from jax import lax
from jax.experimental import pallas as pl
from jax.experimental.pallas import tpu as pltpu
from jax.experimental.pallas import tpu_sc as plsc

B = 16384
DNB = 32
EMB = 128
KSEL = 16
NC, NS, L = 2, 16, 16
NW = NC * NS
NPW = B // NW
SCN = 32
SUB = 4
NSUB = SCN // SUB
NSC = NPW // SCN


def _sc_body(nodes_hbm, neigh_hbm, tv_hbm, tt_hbm, xv_hbm, xt_hbm,
             nid, nidx, cen_v, cen_t, nb_v0, nb_t0, nb_v1, nb_t1,
             out_v, out_t, sem_c, sem_v0, sem_t0, sem_v1, sem_t1):
  cid = lax.axis_index("c")
  sid = lax.axis_index("s")
  wid = cid * NS + sid
  base_w = wid * NPW
  iot = lax.iota(jnp.int32, L)
  zero = jnp.zeros((L,), jnp.float32)
  inv_k = jnp.float32(1.0 / KSEL)

  def rnd_bf16(v, exact=False):
    i = plsc.bitcast(v, jnp.int32)
    if exact:
      lsb = lax.shift_right_logical(i, 16) & 1
      r = (i + 0x7FFF + lsb) & jnp.int32(-65536)
    else:
      r = (i + 0x8000) & jnp.int32(-65536)
    return plsc.bitcast(r, jnp.float32)

  def process_table(nb, cen, out, c):
    crow0 = c * SUB

    for n0 in range(0, SUB, 2):
      us = [[rnd_bf16(cen[crow0 + n0 + p, pl.ds(j * L, L)], exact=True)
             for j in range(8)] for p in range(2)]

      def k_body(k, sims, us=us, n0=n0):
        sims = list(sims)
        m = iot == k
        for p in range(2):
          for g in range(2):
            r = (n0 + p) * DNB + g * L + k
            ms = [us[p][j] * rnd_bf16(nb[r, pl.ds(j * L, L)])
                  for j in range(8)]
            t = (((ms[0] + ms[1]) + (ms[2] + ms[3]))
                 + ((ms[4] + ms[5]) + (ms[6] + ms[7])))
            for c in (8, 4, 2, 1):
              t = t + t.at[iot ^ c].get(mode="promise_in_bounds")
            sims[p * 2 + g] = jnp.where(m, t, sims[p * 2 + g])
        return tuple(sims)

      accs = lax.fori_loop(0, L, k_body, (zero,) * 4)

      for p in range(2):
        n = n0 + p
        s0, i0 = plsc.sort_key_val(accs[p * 2], iot)
        s1, i1 = plsc.sort_key_val(accs[p * 2 + 1], iot + L, descending=True)
        selrows = jnp.where(s0 >= s1, i0, i1) + n * DNB
        crow = crow0 + n
        acc = [zero] * 8
        for s in range(L):
          r_s = selrows[s]
          for j in range(8):
            acc[j] = acc[j] + nb[r_s, pl.ds(j * L, L)]
        for j in range(8):
          out[crow, pl.ds(j * L, L)] = (cen[crow, pl.ds(j * L, L)]
                                        + acc[j] * inv_k)

  def issue(c, nbv, nbt, semv, semt):
    idx = nidx.at[pl.ds(c * SUB * DNB, SUB * DNB)]
    gv = pltpu.async_copy(tv_hbm.at[idx], nbv, semv)
    gt = pltpu.async_copy(tt_hbm.at[idx], nbt, semt)
    return gv, gt

  def sc_loop(sc, _):
    nbase = base_w + sc * SCN
    pltpu.sync_copy(nodes_hbm.at[pl.ds(nbase, SCN)], nid)
    cv = pltpu.async_copy(tv_hbm.at[nid], cen_v, sem_c)
    ct = pltpu.async_copy(tt_hbm.at[nid], cen_t, sem_c)
    pltpu.sync_copy(neigh_hbm.at[pl.ds(nbase * DNB, SCN * DNB)], nidx)
    g0 = issue(0, nb_v0, nb_t0, sem_v0, sem_t0)
    cv.wait()
    ct.wait()

    def pair_loop(i, _):
      c0 = i * 2
      g1 = issue(c0 + 1, nb_v1, nb_t1, sem_v1, sem_t1)
      pltpu.make_async_copy(tv_hbm.at[nidx.at[pl.ds(0, SUB * DNB)]],
                            nb_v0, sem_v0).wait()
      pltpu.make_async_copy(tt_hbm.at[nidx.at[pl.ds(0, SUB * DNB)]],
                            nb_t0, sem_t0).wait()
      process_table(nb_v0, cen_v, out_v, c0)
      process_table(nb_t0, cen_t, out_t, c0)

      @pl.when(i < NSUB // 2 - 1)
      def _():
        issue(c0 + 2, nb_v0, nb_t0, sem_v0, sem_t0)

      pltpu.make_async_copy(tv_hbm.at[nidx.at[pl.ds(0, SUB * DNB)]],
                            nb_v1, sem_v1).wait()
      pltpu.make_async_copy(tt_hbm.at[nidx.at[pl.ds(0, SUB * DNB)]],
                            nb_t1, sem_t1).wait()
      process_table(nb_v1, cen_v, out_v, c0 + 1)
      process_table(nb_t1, cen_t, out_t, c0 + 1)
      return 0

    lax.fori_loop(0, NSUB // 2, pair_loop, 0)
    pltpu.sync_copy(out_v, xv_hbm.at[pl.ds(nbase, SCN)])
    pltpu.sync_copy(out_t, xt_hbm.at[pl.ds(nbase, SCN)])
    return 0

  lax.fori_loop(0, NSC, sc_loop, 0)


def _sc_call(nodes, neigh_flat, tv, tt):
  mesh = plsc.VectorSubcoreMesh(core_axis_name="c", subcore_axis_name="s")
  f = pl.kernel(
      _sc_body,
      out_type=[jax.ShapeDtypeStruct((B, EMB), jnp.float32),
                jax.ShapeDtypeStruct((B, EMB), jnp.float32)],
      mesh=mesh,
      compiler_params=pltpu.CompilerParams(needs_layout_passes=False),
      scratch_types=[
          pltpu.VMEM((SCN,), jnp.int32),
          pltpu.VMEM((SCN * DNB,), jnp.int32),
          pltpu.VMEM((SCN, EMB), jnp.float32),
          pltpu.VMEM((SCN, EMB), jnp.float32),
          pltpu.VMEM((SUB * DNB, EMB), jnp.float32),
          pltpu.VMEM((SUB * DNB, EMB), jnp.float32),
          pltpu.VMEM((SUB * DNB, EMB), jnp.float32),
          pltpu.VMEM((SUB * DNB, EMB), jnp.float32),
          pltpu.VMEM((SCN, EMB), jnp.float32),
          pltpu.VMEM((SCN, EMB), jnp.float32),
          pltpu.SemaphoreType.DMA,
          pltpu.SemaphoreType.DMA,
          pltpu.SemaphoreType.DMA,
          pltpu.SemaphoreType.DMA,
          pltpu.SemaphoreType.DMA,
      ])
  return f(nodes, neigh_flat, tv, tt)


def _tc_body(xv_ref, xt_ref, w_ref, b_ref, o_ref):
  w = w_ref[...]
  bb = b_ref[...]
  ov = jnp.dot(xv_ref[...], w, preferred_element_type=jnp.float32) + bb
  ot = jnp.dot(xt_ref[...], w, preferred_element_type=jnp.float32) + bb
  o_ref[...] = jnp.maximum(ov, 0.0) + jnp.maximum(ot, 0.0)


def _tc_call(xv, xt, W, b):
  BR = 2048
  return pl.pallas_call(
      _tc_body,
      grid=(B // BR,),
      in_specs=[pl.BlockSpec((BR, EMB), lambda i: (i, 0)),
                pl.BlockSpec((BR, EMB), lambda i: (i, 0)),
                pl.BlockSpec((EMB, EMB), lambda i: (0, 0)),
                pl.BlockSpec((1, EMB), lambda i: (0, 0))],
      out_specs=pl.BlockSpec((BR, EMB), lambda i: (i, 0)),
      out_shape=jax.ShapeDtypeStruct((B, EMB), jnp.float32),
  )(xv, xt, W, b.reshape(1, EMB))


def kernel(nodes, neigh, u2e_visual_weight, u2e_text_weight, W, b):
  nodes32 = nodes.astype(jnp.int32)
  neigh_flat = neigh.reshape(-1).astype(jnp.int32)
  xv, xt = _sc_call(nodes32, neigh_flat,
                    u2e_visual_weight, u2e_text_weight)
  return _tc_call(xv, xt, W, b)


# speedup vs baseline: 1.1751x; 1.1751x over previous
import jax
import jax.numpy as jnp
from jax import lax
from jax.experimental import pallas as pl
from jax.experimental.pallas import tpu as pltpu
from jax.experimental.pallas import tpu_sc as plsc

B = 16384
DNB = 32
EMB = 128
KSEL = 16
NC, NS, L = 2, 16, 16
NW = NC * NS
NPW = B // NW
SCN = 32
SUB = 4
NSUB = SCN // SUB
NSC = NPW // SCN


def _sc_body(nodes_hbm, neigh_hbm, tv_hbm, tt_hbm, xv_hbm, xt_hbm,
             nid, nidx, cen_v, cen_t, nb_v0, nb_t0, nb_v1, nb_t1,
             out_v, out_t, sem_c, sem_v0, sem_t0, sem_v1, sem_t1):
  cid = lax.axis_index("c")
  sid = lax.axis_index("s")
  wid = cid * NS + sid
  base_w = wid * NPW
  iot = lax.iota(jnp.int32, L)
  zero = jnp.zeros((L,), jnp.float32)
  inv_k = jnp.float32(1.0 / KSEL)

  def rnd_bf16(v, exact=False):
    i = plsc.bitcast(v, jnp.int32)
    if exact:
      lsb = lax.shift_right_logical(i, 16) & 1
      r = (i + 0x7FFF + lsb) & jnp.int32(-65536)
    else:
      r = (i + 0x8000) & jnp.int32(-65536)
    return plsc.bitcast(r, jnp.float32)

  def process_table(nb, cen, out, c):
    crow0 = c * SUB

    for n0 in range(0, SUB, 2):
      us = [[rnd_bf16(cen[crow0 + n0 + p, pl.ds(j * L, L)], exact=True)
             for j in range(8)] for p in range(2)]

      def k_body(k, sims, us=us, n0=n0):
        sims = list(sims)
        m = iot == k
        for p in range(2):
          for g in range(2):
            r = (n0 + p) * DNB + g * L + k
            rs = []
            for j in range(0, 8, 2):
              ab = plsc.pack(nb[r, pl.ds(j * L, L)], nb[r, pl.ds((j + 1) * L, L)],
                             format=plsc.PackFormat.INTERLEAVED)
              a, b2 = plsc.unpack(ab, format=plsc.PackFormat.INTERLEAVED)
              rs += [a, b2]
            ms = [us[p][j] * rs[j] for j in range(8)]
            t = (((ms[0] + ms[1]) + (ms[2] + ms[3]))
                 + ((ms[4] + ms[5]) + (ms[6] + ms[7])))
            d = jnp.sum(t)
            sims[p * 2 + g] = jnp.where(m, d, sims[p * 2 + g])
        return tuple(sims)

      accs = lax.fori_loop(0, L, k_body, (zero,) * 4)

      for p in range(2):
        n = n0 + p
        s0, i0 = plsc.sort_key_val(accs[p * 2], iot)
        s1, i1 = plsc.sort_key_val(accs[p * 2 + 1], iot + L, descending=True)
        selrows = jnp.where(s0 >= s1, i0, i1) + n * DNB
        crow = crow0 + n
        acc = [zero] * 8
        for s in range(L):
          r_s = selrows[s]
          for j in range(8):
            acc[j] = acc[j] + nb[r_s, pl.ds(j * L, L)]
        for j in range(8):
          out[crow, pl.ds(j * L, L)] = (cen[crow, pl.ds(j * L, L)]
                                        + acc[j] * inv_k)

  def issue(c, nbv, nbt, semv, semt):
    idx = nidx.at[pl.ds(c * SUB * DNB, SUB * DNB)]
    gv = pltpu.async_copy(tv_hbm.at[idx], nbv, semv)
    gt = pltpu.async_copy(tt_hbm.at[idx], nbt, semt)
    return gv, gt

  def sc_loop(sc, _):
    nbase = base_w + sc * SCN
    pltpu.sync_copy(nodes_hbm.at[pl.ds(nbase, SCN)], nid)
    cv = pltpu.async_copy(tv_hbm.at[nid], cen_v, sem_c)
    ct = pltpu.async_copy(tt_hbm.at[nid], cen_t, sem_c)
    pltpu.sync_copy(neigh_hbm.at[pl.ds(nbase * DNB, SCN * DNB)], nidx)
    g0 = issue(0, nb_v0, nb_t0, sem_v0, sem_t0)
    cv.wait()
    ct.wait()

    def pair_loop(i, _):
      c0 = i * 2
      g1 = issue(c0 + 1, nb_v1, nb_t1, sem_v1, sem_t1)
      pltpu.make_async_copy(tv_hbm.at[nidx.at[pl.ds(0, SUB * DNB)]],
                            nb_v0, sem_v0).wait()
      pltpu.make_async_copy(tt_hbm.at[nidx.at[pl.ds(0, SUB * DNB)]],
                            nb_t0, sem_t0).wait()
      process_table(nb_v0, cen_v, out_v, c0)
      process_table(nb_t0, cen_t, out_t, c0)

      @pl.when(i < NSUB // 2 - 1)
      def _():
        issue(c0 + 2, nb_v0, nb_t0, sem_v0, sem_t0)

      pltpu.make_async_copy(tv_hbm.at[nidx.at[pl.ds(0, SUB * DNB)]],
                            nb_v1, sem_v1).wait()
      pltpu.make_async_copy(tt_hbm.at[nidx.at[pl.ds(0, SUB * DNB)]],
                            nb_t1, sem_t1).wait()
      process_table(nb_v1, cen_v, out_v, c0 + 1)
      process_table(nb_t1, cen_t, out_t, c0 + 1)
      return 0

    lax.fori_loop(0, NSUB // 2, pair_loop, 0)
    pltpu.sync_copy(out_v, xv_hbm.at[pl.ds(nbase, SCN)])
    pltpu.sync_copy(out_t, xt_hbm.at[pl.ds(nbase, SCN)])
    return 0

  lax.fori_loop(0, NSC, sc_loop, 0)


def _sc_call(nodes, neigh_flat, tv, tt):
  mesh = plsc.VectorSubcoreMesh(core_axis_name="c", subcore_axis_name="s")
  f = pl.kernel(
      _sc_body,
      out_type=[jax.ShapeDtypeStruct((B, EMB), jnp.float32),
                jax.ShapeDtypeStruct((B, EMB), jnp.float32)],
      mesh=mesh,
      compiler_params=pltpu.CompilerParams(needs_layout_passes=False),
      scratch_types=[
          pltpu.VMEM((SCN,), jnp.int32),
          pltpu.VMEM((SCN * DNB,), jnp.int32),
          pltpu.VMEM((SCN, EMB), jnp.float32),
          pltpu.VMEM((SCN, EMB), jnp.float32),
          pltpu.VMEM((SUB * DNB, EMB), jnp.float32),
          pltpu.VMEM((SUB * DNB, EMB), jnp.float32),
          pltpu.VMEM((SUB * DNB, EMB), jnp.float32),
          pltpu.VMEM((SUB * DNB, EMB), jnp.float32),
          pltpu.VMEM((SCN, EMB), jnp.float32),
          pltpu.VMEM((SCN, EMB), jnp.float32),
          pltpu.SemaphoreType.DMA,
          pltpu.SemaphoreType.DMA,
          pltpu.SemaphoreType.DMA,
          pltpu.SemaphoreType.DMA,
          pltpu.SemaphoreType.DMA,
      ])
  return f(nodes, neigh_flat, tv, tt)


def _tc_body(xv_ref, xt_ref, w_ref, b_ref, o_ref):
  w = w_ref[...]
  bb = b_ref[...]
  ov = jnp.dot(xv_ref[...], w, preferred_element_type=jnp.float32) + bb
  ot = jnp.dot(xt_ref[...], w, preferred_element_type=jnp.float32) + bb
  o_ref[...] = jnp.maximum(ov, 0.0) + jnp.maximum(ot, 0.0)


def _tc_call(xv, xt, W, b):
  BR = 2048
  return pl.pallas_call(
      _tc_body,
      grid=(B // BR,),
      in_specs=[pl.BlockSpec((BR, EMB), lambda i: (i, 0)),
                pl.BlockSpec((BR, EMB), lambda i: (i, 0)),
                pl.BlockSpec((EMB, EMB), lambda i: (0, 0)),
                pl.BlockSpec((1, EMB), lambda i: (0, 0))],
      out_specs=pl.BlockSpec((BR, EMB), lambda i: (i, 0)),
      out_shape=jax.ShapeDtypeStruct((B, EMB), jnp.float32),
  )(xv, xt, W, b.reshape(1, EMB))


def kernel(nodes, neigh, u2e_visual_weight, u2e_text_weight, W, b):
  nodes32 = nodes.astype(jnp.int32)
  neigh_flat = neigh.reshape(-1).astype(jnp.int32)
  xv, xt = _sc_call(nodes32, neigh_flat,
                    u2e_visual_weight, u2e_text_weight)
  return _tc_call(xv, xt, W, b)
